# baseline (speedup 1.0000x reference)
import functools

import jax
import jax.numpy as jnp
from jax import lax
from jax.experimental import pallas as pl
from jax.experimental.pallas import tpu as pltpu
from jax.experimental.pallas import tpu_sc as plsc

_B = 16384
_L = 200
_D = 32
_N = _B * _L

_NC = 2
_NS = 16
_NW = _NC * _NS

_F = 4
_CD = _F * _D
_NCOMP = 6 ** _F
_N4 = _N // _F

_SUB = 128
_K = 2
_CSUB = _SUB * _K
_CHUNK = _CSUB * _F
_S = 128
_ROWS_PER_W = _N4 // _NW
_N_CHUNKS = _ROWS_PER_W // _CSUB


@functools.cache
def _build():
    mesh = plsc.VectorSubcoreMesh(core_axis_name="c", subcore_axis_name="s")

    @functools.partial(
        pl.kernel,
        mesh=mesh,
        compiler_params=pltpu.CompilerParams(use_tc_tiling_on_sc=False,
                                             needs_layout_passes=False),
        out_type=jax.ShapeDtypeStruct((_N4, _CD), jnp.float32),
        scratch_types=[
            pltpu.VMEM((_CHUNK,), jnp.int32),
            pltpu.VMEM((_CHUNK,), jnp.int32),
            pltpu.VMEM((_CSUB,), jnp.int32),
            pltpu.VMEM((_CSUB,), jnp.int32),
            pltpu.VMEM((_CSUB, _CD), jnp.float32),
            pltpu.VMEM((_CSUB, _CD), jnp.float32),
            pltpu.VMEM((6 * _D,), jnp.float32),
            pltpu.VMEM_SHARED((_NCOMP, _CD), jnp.float32),
            pltpu.SemaphoreType.DMA,
            pltpu.SemaphoreType.DMA,
            pltpu.SemaphoreType.DMA,
            pltpu.SemaphoreType.DMA,
            pltpu.SemaphoreType.DMA,
            pltpu.SemaphoreType.DMA,
        ],
    )
    def emb(idx_hbm, ctable_hbm, table_hbm, out_hbm, idx0, idx1, cidx0, cidx1,
            rows0, rows1, table_v, ct_v, isem0, isem1, gsem0, gsem1, osem0,
            osem1):
        wid = lax.axis_index("s") * _NC + lax.axis_index("c")
        crow0 = wid * _ROWS_PER_W

        @pl.when(lax.axis_index("s") == 0)
        def _():
            pltpu.sync_copy(ctable_hbm, ct_v)
        pltpu.sync_copy(table_hbm, table_v)
        plsc.subcore_barrier()

        def cbase(c):
            return pl.multiple_of(crow0 + c * _CSUB, _CSUB)

        def idx_src(c):
            return idx_hbm.at[pl.ds(pl.multiple_of(cbase(c) * _F, _CHUNK),
                                    _CHUNK)]

        def out_dst(c):
            return out_hbm.at[pl.ds(cbase(c), _CSUB)]

        lanes = lax.iota(jnp.int32, 16)

        def compute_comp(idxb, cidxb):
            for g in range(_S // 16):
                base = lanes * _F + g * 64
                a0 = plsc.load_gather(idxb, [base])
                a1 = plsc.load_gather(idxb, [base + 1])
                a2 = plsc.load_gather(idxb, [base + 2])
                a3 = plsc.load_gather(idxb, [base + 3])
                cidxb[pl.ds(g * 16, 16)] = ((a0 * 6 + a1) * 6 + a2) * 6 + a3

        def fire_gathers(cidxb, rowsb, gsem):
            for j in range(_S // _SUB):
                pltpu.async_copy(ct_v.at[cidxb.at[pl.ds(j * _SUB, _SUB)]],
                                 rowsb.at[pl.ds(j * _SUB, _SUB)], gsem)

        def wait_gathers(cidxb, rowsb, gsem):
            for j in range(_S // _SUB):
                pltpu.make_async_copy(ct_v.at[cidxb.at[pl.ds(j * _SUB, _SUB)]],
                                      rowsb.at[pl.ds(j * _SUB, _SUB)],
                                      gsem).wait()

        def expand_vpu(idxb, rowsb):
            def gbody(g, carry):
                rvec = lanes + (_S + g * 16)
                posbase = rvec * _F
                for m in range(_F):
                    am = plsc.load_gather(idxb, [posbase + m])
                    tb = am * _D
                    for cc in range(_D):
                        val = plsc.load_gather(table_v, [tb + cc])
                        col = jnp.full((16,), m * _D + cc, jnp.int32)
                        plsc.store_scatter(rowsb, [rvec, col], val)
                return carry
            lax.fori_loop(0, (_CSUB - _S) // 16, gbody, 0)

        def wait_copy(src, dst, sem):
            pltpu.make_async_copy(src, dst, sem).wait()

        pltpu.async_copy(idx_src(0), idx0, isem0)
        pltpu.async_copy(idx_src(1), idx1, isem1)
        wait_copy(idx_src(0), idx0, isem0)
        compute_comp(idx0, cidx0)
        fire_gathers(cidx0, rows0, gsem0)

        def body(t, carry):
            c = t * 2
            expand_vpu(idx0, rows0)
            wait_gathers(cidx0, rows0, gsem0)
            pltpu.async_copy(rows0, out_dst(c), osem0)

            @pl.when(c + 2 < _N_CHUNKS)
            def _():
                pltpu.async_copy(idx_src(c + 2), idx0, isem0)

            wait_copy(idx_src(c + 1), idx1, isem1)
            compute_comp(idx1, cidx1)

            @pl.when(t > 0)
            def _():
                wait_copy(rows1, out_dst(c - 1), osem1)

            fire_gathers(cidx1, rows1, gsem1)

            expand_vpu(idx1, rows1)
            wait_gathers(cidx1, rows1, gsem1)
            pltpu.async_copy(rows1, out_dst(c + 1), osem1)

            @pl.when(c + 3 < _N_CHUNKS)
            def _():
                pltpu.async_copy(idx_src(c + 3), idx1, isem1)

            @pl.when(c + 2 < _N_CHUNKS)
            def _():
                wait_copy(idx_src(c + 2), idx0, isem0)
                compute_comp(idx0, cidx0)
                wait_copy(rows0, out_dst(c), osem0)
                fire_gathers(cidx0, rows0, gsem0)

            return carry

        lax.fori_loop(0, _N_CHUNKS // 2, body, 0)

        wait_copy(rows0, out_dst(_N_CHUNKS - 2), osem0)
        wait_copy(rows1, out_dst(_N_CHUNKS - 1), osem1)

    return emb


def kernel(action, table):
    idx = action.reshape(_N)
    combos = jnp.arange(_NCOMP)
    ctable = jnp.concatenate(
        [jnp.take(table, (combos // (6 ** (_F - 1 - m))) % 6, axis=0)
         for m in range(_F)], axis=1)
    rows = _build()(idx, ctable, table.reshape(6 * _D))
    return rows.reshape(_B, _L * _D)


# speedup vs baseline: 3.9204x; 3.9204x over previous
import functools

import jax
import jax.numpy as jnp
from jax import lax
from jax.experimental import pallas as pl
from jax.experimental.pallas import tpu as pltpu
from jax.experimental.pallas import tpu_sc as plsc

_B = 16384
_L = 200
_D = 32
_N = _B * _L

_NC = 2
_NS = 16
_NW = _NC * _NS

_F = 4
_CD = _F * _D
_NCOMP = 6 ** _F
_N4 = _N // _F

_SUB = 128
_K = 2
_CSUB = _SUB * _K
_CHUNK = _CSUB * _F
_S = 192
_ROWS_PER_W = _N4 // _NW
_N_CHUNKS = _ROWS_PER_W // _CSUB


@functools.cache
def _build():
    mesh = plsc.VectorSubcoreMesh(core_axis_name="c", subcore_axis_name="s")

    @functools.partial(
        pl.kernel,
        mesh=mesh,
        compiler_params=pltpu.CompilerParams(use_tc_tiling_on_sc=False,
                                             needs_layout_passes=False),
        out_type=jax.ShapeDtypeStruct((_N4, _CD), jnp.float32),
        scratch_types=[
            pltpu.VMEM((_CHUNK,), jnp.int32),
            pltpu.VMEM((_CHUNK,), jnp.int32),
            pltpu.VMEM((_CSUB,), jnp.int32),
            pltpu.VMEM((_CSUB,), jnp.int32),
            pltpu.VMEM((_CSUB, _CD), jnp.float32),
            pltpu.VMEM((_CSUB, _CD), jnp.float32),
            pltpu.VMEM((6 * _D,), jnp.float32),
            pltpu.VMEM_SHARED((_NCOMP, _CD), jnp.float32),
            pltpu.SemaphoreType.DMA,
            pltpu.SemaphoreType.DMA,
            pltpu.SemaphoreType.DMA,
            pltpu.SemaphoreType.DMA,
            pltpu.SemaphoreType.DMA,
            pltpu.SemaphoreType.DMA,
        ],
    )
    def emb(idx_hbm, ctable_hbm, table_hbm, out_hbm, idx0, idx1, cidx0, cidx1,
            rows0, rows1, table_v, ct_v, isem0, isem1, gsem0, gsem1, osem0,
            osem1):
        wid = lax.axis_index("s") * _NC + lax.axis_index("c")
        crow0 = wid * _ROWS_PER_W

        @pl.when(lax.axis_index("s") == 0)
        def _():
            pltpu.sync_copy(ctable_hbm, ct_v)
        pltpu.sync_copy(table_hbm, table_v)
        plsc.subcore_barrier()

        def cbase(c):
            return pl.multiple_of(crow0 + c * _CSUB, _CSUB)

        def idx_src(c):
            return idx_hbm.at[pl.ds(pl.multiple_of(cbase(c) * _F, _CHUNK),
                                    _CHUNK)]

        def out_dst(c):
            return out_hbm.at[pl.ds(cbase(c), _CSUB)]

        lanes = lax.iota(jnp.int32, 16)

        def compute_comp(idxb, cidxb):
            for g in range(_S // 16):
                base = lanes * _F + g * 64
                a0 = plsc.load_gather(idxb, [base])
                a1 = plsc.load_gather(idxb, [base + 1])
                a2 = plsc.load_gather(idxb, [base + 2])
                a3 = plsc.load_gather(idxb, [base + 3])
                cidxb[pl.ds(g * 16, 16)] = ((a0 * 6 + a1) * 6 + a2) * 6 + a3

        def fire_gathers(cidxb, rowsb, gsem):
            for j in range(_S // _SUB):
                pltpu.async_copy(ct_v.at[cidxb.at[pl.ds(j * _SUB, _SUB)]],
                                 rowsb.at[pl.ds(j * _SUB, _SUB)], gsem)

        def wait_gathers(cidxb, rowsb, gsem):
            for j in range(_S // _SUB):
                pltpu.make_async_copy(ct_v.at[cidxb.at[pl.ds(j * _SUB, _SUB)]],
                                      rowsb.at[pl.ds(j * _SUB, _SUB)],
                                      gsem).wait()

        def expand_vpu(idxb, rowsb):
            @functools.partial(plsc.parallel_loop, 0, (_CSUB - _S) // 16,
                               unroll=2)
            def _gbody(g):
                rvec = lanes + (_S + g * 16)
                posbase = rvec * _F
                for m in range(_F):
                    am = plsc.load_gather(idxb, [posbase + m])
                    tb = am * _D
                    for cc in range(_D):
                        val = plsc.load_gather(table_v, [tb + cc])
                        col = jnp.full((16,), m * _D + cc, jnp.int32)
                        plsc.store_scatter(rowsb, [rvec, col], val)

        def wait_copy(src, dst, sem):
            pltpu.make_async_copy(src, dst, sem).wait()

        pltpu.async_copy(idx_src(0), idx0, isem0)
        pltpu.async_copy(idx_src(1), idx1, isem1)
        wait_copy(idx_src(0), idx0, isem0)
        compute_comp(idx0, cidx0)
        fire_gathers(cidx0, rows0, gsem0)

        def body(t, carry):
            c = t * 2
            expand_vpu(idx0, rows0)
            wait_gathers(cidx0, rows0, gsem0)
            pltpu.async_copy(rows0, out_dst(c), osem0)

            @pl.when(c + 2 < _N_CHUNKS)
            def _():
                pltpu.async_copy(idx_src(c + 2), idx0, isem0)

            wait_copy(idx_src(c + 1), idx1, isem1)
            compute_comp(idx1, cidx1)

            @pl.when(t > 0)
            def _():
                wait_copy(rows1, out_dst(c - 1), osem1)

            fire_gathers(cidx1, rows1, gsem1)

            expand_vpu(idx1, rows1)
            wait_gathers(cidx1, rows1, gsem1)
            pltpu.async_copy(rows1, out_dst(c + 1), osem1)

            @pl.when(c + 3 < _N_CHUNKS)
            def _():
                pltpu.async_copy(idx_src(c + 3), idx1, isem1)

            @pl.when(c + 2 < _N_CHUNKS)
            def _():
                wait_copy(idx_src(c + 2), idx0, isem0)
                compute_comp(idx0, cidx0)
                wait_copy(rows0, out_dst(c), osem0)
                fire_gathers(cidx0, rows0, gsem0)

            return carry

        lax.fori_loop(0, _N_CHUNKS // 2, body, 0)

        wait_copy(rows0, out_dst(_N_CHUNKS - 2), osem0)
        wait_copy(rows1, out_dst(_N_CHUNKS - 1), osem1)

    return emb


def kernel(action, table):
    idx = action.reshape(_N)
    combos = jnp.arange(_NCOMP)
    ctable = jnp.concatenate(
        [jnp.take(table, (combos // (6 ** (_F - 1 - m))) % 6, axis=0)
         for m in range(_F)], axis=1)
    rows = _build()(idx, ctable, table.reshape(6 * _D))
    return rows.reshape(_B, _L * _D)
